# baseline (speedup 1.0000x reference)
import functools

import jax
import jax.numpy as jnp
from jax import lax
from jax.experimental import pallas as pl
from jax.experimental.pallas import tpu as pltpu
from jax.experimental.pallas import tpu_sc as plsc

N = 2048
RB = 256
G = N // RB
E = 131072
_HI = lax.Precision.HIGHEST

TILE_ROWS = 64
CH = 16384
HALF = N // 2
UNROLL = 8


def _pack_body(e0_ref, e1_ref, o0_ref, o1_ref):
    o0_ref[...] = jnp.left_shift(e0_ref[0:1, :], 11) | e0_ref[1:2, :]
    o1_ref[...] = jnp.left_shift(e1_ref[0:1, :], 11) | e1_ref[1:2, :]


def _pack_edges(ei0, ei1):
    out = (jax.ShapeDtypeStruct((1, E), jnp.int32),
           jax.ShapeDtypeStruct((1, E), jnp.int32))
    return pl.pallas_call(_pack_body, out_shape=out)(ei0, ei1)


def _adj_sc_body(src, dst, acc, buf0, buf1, sem0, sem1):
    cid = lax.axis_index("c")
    sid = lax.axis_index("s")
    wid = sid * 2 + cid
    r0 = wid * TILE_ROWS
    ones = jnp.ones((16,), jnp.float32)
    zeros16 = jnp.zeros((16,), jnp.float32)
    nchunks = E // CH
    bufs = (buf0, buf1)
    sems = (sem0, sem1)
    if True:
        for half in range(2):
            c0 = half * HALF

            def zrow(i, carry):
                for k in range(HALF // 16):
                    acc[i, pl.ds(k * 16, 16)] = zeros16
                return carry

            lax.fori_loop(0, TILE_ROWS, zrow, 0)

            copies = [
                pltpu.async_copy(src.at[pl.ds(0, CH)], buf0, sem0),
                pltpu.async_copy(src.at[pl.ds(CH, CH)], buf1, sem1),
            ]
            for t in range(nchunks):
                b = t % 2
                copies[b].wait()
                buf = bufs[b]

                def scan_body(i, buf=buf, half=half):
                    fv = buf[pl.ds(i * 16, 16)]
                    m = lax.shift_right_logical(fv, 17) == wid
                    cl = fv & (N - 1)
                    if half == 0:
                        m = m & (cl < HALF)
                    else:
                        m = m & (cl >= HALF)
                    ridx = lax.shift_right_logical(fv, 11) & (TILE_ROWS - 1)
                    cidx = fv & (HALF - 1)
                    plsc.addupdate_scatter(acc, [ridx, cidx], ones, mask=m)

                plsc.parallel_loop(0, CH // 16, unroll=UNROLL)(scan_body)
                if t + 2 < nchunks:
                    copies[b] = pltpu.async_copy(
                        src.at[pl.ds((t + 2) * CH, CH)], buf, sems[b])
            pltpu.sync_copy(acc, dst.at[pl.ds(r0, TILE_ROWS), pl.ds(c0, HALF)])


def _adj_sc(flat):
    mesh = plsc.VectorSubcoreMesh(core_axis_name="c", subcore_axis_name="s")
    f = functools.partial(
        pl.kernel,
        out_type=jax.ShapeDtypeStruct((N, N), jnp.float32),
        mesh=mesh,
        scratch_types=[
            pltpu.VMEM((TILE_ROWS, HALF), jnp.float32),
            pltpu.VMEM((CH,), jnp.int32),
            pltpu.VMEM((CH,), jnp.int32),
            pltpu.SemaphoreType.DMA,
            pltpu.SemaphoreType.DMA,
        ],
        compiler_params=pltpu.CompilerParams(use_tc_tiling_on_sc=False,
                                             needs_layout_passes=False),
    )(_adj_sc_body)
    return f(flat)


def _deg_body(a_ref, do_ref, di_ref):
    a = a_ref[...]
    do_ref[...] = lax.rsqrt(jnp.clip(jnp.sum(a, axis=1, keepdims=True), 1.0, None))
    di_ref[...] = lax.rsqrt(jnp.clip(jnp.sum(a, axis=0, keepdims=True), 1.0, None))


def _degrees(a):
    out = (jax.ShapeDtypeStruct((N, 1), jnp.float32),
           jax.ShapeDtypeStruct((1, N), jnp.float32))
    return pl.pallas_call(_deg_body, out_shape=out)(a)


def _stack_body(prec, a_ref, x_ref, dos_ref, dis_ref, w0_ref, b0_ref, w1_ref,
                b1_ref, o_ref, ot_ref, h1_scr):
    p = pl.program_id(0)
    j = pl.program_id(1)

    @pl.when(p == 0)
    def _():
        hs = x_ref[...] * dos_ref[...]
        agg = lax.dot_general(a_ref[...], hs, (((0,), (0,)), ((), ())),
                              preferred_element_type=jnp.float32, precision=prec)
        agg = agg * dis_ref[pl.ds(j * RB, RB), :]
        o = jnp.dot(agg, w0_ref[...], preferred_element_type=jnp.float32,
                    precision=prec) + b0_ref[...]
        h1_scr[pl.ds(j * RB, RB), :] = jnp.maximum(o, 0.0)

    @pl.when(p == 1)
    def _():
        hs = h1_scr[...] * dos_ref[...]
        agg = lax.dot_general(a_ref[...], hs, (((0,), (0,)), ((), ())),
                              preferred_element_type=jnp.float32, precision=prec)
        agg = agg * dis_ref[pl.ds(j * RB, RB), :]
        o = jnp.dot(agg, w1_ref[...], preferred_element_type=jnp.float32,
                    precision=prec) + b1_ref[...]
        o_ref[...] = o
        ot_ref[...] = o.T


def _gcn_stack(a, x, dos, dis, w0, b0, w1, b1, prec=_HI):
    hin = x.shape[1]
    hmid = w0.shape[1]
    hout = w1.shape[1]
    return pl.pallas_call(
        functools.partial(_stack_body, prec),
        grid=(2, G),
        in_specs=[
            pl.BlockSpec((N, RB), lambda p, j: (0, j)),
            pl.BlockSpec((N, hin), lambda p, j: (0, 0)),
            pl.BlockSpec((N, 1), lambda p, j: (0, 0)),
            pl.BlockSpec((N, 1), lambda p, j: (0, 0)),
            pl.BlockSpec((hin, hmid), lambda p, j: (0, 0)),
            pl.BlockSpec((1, hmid), lambda p, j: (0, 0)),
            pl.BlockSpec((hmid, hout), lambda p, j: (0, 0)),
            pl.BlockSpec((1, hout), lambda p, j: (0, 0)),
        ],
        out_specs=(pl.BlockSpec((RB, hout), lambda p, j: (j, 0)),
                   pl.BlockSpec((hout, RB), lambda p, j: (0, j))),
        out_shape=(jax.ShapeDtypeStruct((N, hout), jnp.float32),
                   jax.ShapeDtypeStruct((hout, N), jnp.float32)),
        scratch_shapes=[pltpu.VMEM((N, hmid), jnp.float32)],
    )(a, x, dos, dis, w0, b0.reshape(1, hmid), w1, b1.reshape(1, hout))


def _fuse_body(f0_ref, f1_ref, c0_ref, c1_ref, z_ref):
    zz = (jnp.dot(f0_ref[...], c0_ref[...], preferred_element_type=jnp.float32)
          + jnp.dot(f1_ref[...], c1_ref[...], preferred_element_type=jnp.float32))
    zz = zz - jnp.max(zz, axis=1, keepdims=True)
    e = jnp.exp(zz)
    z_ref[...] = e / jnp.sum(e, axis=1, keepdims=True)


def _fuse(f0, f1, c0, c1):
    return pl.pallas_call(
        _fuse_body,
        out_shape=jax.ShapeDtypeStruct(f0.shape, jnp.float32),
    )(f0, f1, c0, c1)


def _gfn1_body(a0_ref, a1_ref, w_ref, b_ref, t_ref):
    adj = jnp.minimum(a0_ref[...], 1.0) + jnp.minimum(a1_ref[...], 1.0)
    t = jnp.dot(adj, w_ref[...], preferred_element_type=jnp.float32) + b_ref[...]
    t_ref[...] = jnp.maximum(t, 0.0)


def _gfn1(a0, a1, w, b):
    return pl.pallas_call(
        _gfn1_body,
        grid=(G,),
        in_specs=[
            pl.BlockSpec((RB, N), lambda j: (j, 0)),
            pl.BlockSpec((RB, N), lambda j: (j, 0)),
            pl.BlockSpec((N, N // 2), lambda j: (0, 0)),
            pl.BlockSpec((1, N // 2), lambda j: (0, 0)),
        ],
        out_specs=pl.BlockSpec((RB, N // 2), lambda j: (j, 0)),
        out_shape=jax.ShapeDtypeStruct((N, N // 2), jnp.float32),
    )(a0, a1, w, b.reshape(1, N // 2))


def _gfn2_body(t_ref, w_ref, b_ref, o_ref):
    v = jnp.dot(t_ref[...], w_ref[...], preferred_element_type=jnp.float32) + b_ref[...]
    o_ref[...] = jnp.where(v > jnp.float32(0.4), 1.0, 0.0)


def _gfn2(t, w, b):
    return pl.pallas_call(
        _gfn2_body,
        grid=(G,),
        in_specs=[
            pl.BlockSpec((RB, N // 2), lambda j: (j, 0)),
            pl.BlockSpec((N // 2, N), lambda j: (0, 0)),
            pl.BlockSpec((1, N), lambda j: (0, 0)),
        ],
        out_specs=pl.BlockSpec((RB, N), lambda j: (j, 0)),
        out_shape=jax.ShapeDtypeStruct((N, N), jnp.float32),
    )(t, w, b.reshape(1, N))


def _aloop_body(fr_ref, fc_ref, al_ref, d_ref):
    i = pl.program_id(0)
    j = pl.program_id(1)
    sym = fr_ref[...] + fc_ref[...].T
    a = jnp.where(sym != 0.0, 1.0, 0.0)
    row_ids = lax.broadcasted_iota(jnp.int32, (RB, RB), 0) + i * RB
    col_ids = lax.broadcasted_iota(jnp.int32, (RB, RB), 1) + j * RB
    a = a + jnp.where(row_ids == col_ids, 1.0, 0.0)
    al_ref[...] = a

    @pl.when(j == 0)
    def _():
        d_ref[...] = jnp.zeros_like(d_ref)

    d_ref[...] += jnp.sum(a, axis=1, keepdims=True)

    @pl.when(j == G - 1)
    def _():
        d_ref[...] = lax.rsqrt(jnp.clip(d_ref[...], 1.0, None))


def _aloop(fused):
    return pl.pallas_call(
        _aloop_body,
        grid=(G, G),
        in_specs=[
            pl.BlockSpec((RB, RB), lambda i, j: (i, j)),
            pl.BlockSpec((RB, RB), lambda i, j: (j, i)),
        ],
        out_specs=(
            pl.BlockSpec((RB, RB), lambda i, j: (i, j)),
            pl.BlockSpec((RB, 1), lambda i, j: (i, 0)),
        ),
        out_shape=(jax.ShapeDtypeStruct((N, N), jnp.float32),
                   jax.ShapeDtypeStruct((N, 1), jnp.float32)),
    )(fused, fused)


def _dec_body(hb_ref, ht_ref, o_ref):
    o_ref[...] = jnp.dot(hb_ref[...], ht_ref[...],
                         preferred_element_type=jnp.float32)


def _decode(h, ht):
    hd = h.shape[1]
    return pl.pallas_call(
        _dec_body,
        grid=(G,),
        in_specs=[
            pl.BlockSpec((RB, hd), lambda j: (j, 0)),
            pl.BlockSpec((hd, N), lambda j: (0, 0)),
        ],
        out_specs=pl.BlockSpec((RB, N), lambda j: (j, 0)),
        out_shape=jax.ShapeDtypeStruct((N, N), jnp.float32),
    )(h, ht)


def kernel(x0, x1, edge_index0, edge_index1, Wv00, bv00, Wv01, bv01,
           Wv10, bv10, Wv11, bv11, F0, F1, GW1, Gb1, GW2, Gb2,
           Wm0, bm0, Wm1, bm1):
    p0, p1 = _pack_edges(edge_index0, edge_index1)
    a0 = _adj_sc(p0.reshape(E))
    do0, di0 = _degrees(a0)
    f0, _ = _gcn_stack(a0, x0, do0, di0.reshape(N, 1), Wv00, bv00, Wv01, bv01,
                       prec=lax.Precision.DEFAULT)
    a1 = _adj_sc(p1.reshape(E))
    do1, di1 = _degrees(a1)
    f1, _ = _gcn_stack(a1, x1, do1, di1.reshape(N, 1), Wv10, bv10, Wv11, bv11,
                       prec=lax.Precision.DEFAULT)
    t = _gfn1(a0, a1, GW1, Gb1)
    z = _fuse(f0, f1, F0, F1)
    fused = _gfn2(t, GW2, Gb2)
    al, scons = _aloop(fused)
    h, ht = _gcn_stack(al, z, scons, scons, Wm0, bm0, Wm1, bm1,
                       prec=lax.Precision.DEFAULT)
    adj_rec = _decode(h, ht)
    return (fused, adj_rec, h)


# speedup vs baseline: 1.0110x; 1.0110x over previous
import functools

import jax
import jax.numpy as jnp
from jax import lax
from jax.experimental import pallas as pl
from jax.experimental.pallas import tpu as pltpu
from jax.experimental.pallas import tpu_sc as plsc

N = 2048
RB = 256
G = N // RB
E = 131072
_HI = lax.Precision.HIGHEST

TILE_ROWS = 64
CH = 16384
HALF = N // 2
UNROLL = 8


def _pack_body(e0_ref, e1_ref, o0_ref, o1_ref):
    o0_ref[...] = jnp.left_shift(e0_ref[0:1, :], 11) | e0_ref[1:2, :]
    o1_ref[...] = jnp.left_shift(e1_ref[0:1, :], 11) | e1_ref[1:2, :]


def _pack_edges(ei0, ei1):
    out = (jax.ShapeDtypeStruct((1, E), jnp.int32),
           jax.ShapeDtypeStruct((1, E), jnp.int32))
    return pl.pallas_call(_pack_body, out_shape=out)(ei0, ei1)


def _adj_sc_body(e0_hbm, e1_hbm, m0_hbm, m1_hbm, acc, buf0, buf1, sem0, sem1):
    cid = lax.axis_index("c")
    sid = lax.axis_index("s")
    wid = sid * 2 + cid
    r0 = wid * TILE_ROWS
    ones = jnp.ones((16,), jnp.float32)
    zeros16 = jnp.zeros((16,), jnp.float32)
    nchunks = E // CH
    bufs = (buf0, buf1)
    sems = (sem0, sem1)
    for view in range(2):
        src = e0_hbm if view == 0 else e1_hbm
        dst = m0_hbm if view == 0 else m1_hbm
        for half in range(2):
            c0 = half * HALF

            def zrow(i, carry):
                for k in range(HALF // 16):
                    acc[i, pl.ds(k * 16, 16)] = zeros16
                return carry

            lax.fori_loop(0, TILE_ROWS, zrow, 0)

            copies = [
                pltpu.async_copy(src.at[pl.ds(0, CH)], buf0, sem0),
                pltpu.async_copy(src.at[pl.ds(CH, CH)], buf1, sem1),
            ]
            for t in range(nchunks):
                b = t % 2
                copies[b].wait()
                buf = bufs[b]

                def scan_body(i, buf=buf, half=half):
                    fv = buf[pl.ds(i * 16, 16)]
                    m = lax.shift_right_logical(fv, 17) == wid
                    cl = fv & (N - 1)
                    if half == 0:
                        m = m & (cl < HALF)
                    else:
                        m = m & (cl >= HALF)
                    ridx = lax.shift_right_logical(fv, 11) & (TILE_ROWS - 1)
                    cidx = fv & (HALF - 1)
                    plsc.addupdate_scatter(acc, [ridx, cidx], ones, mask=m)

                plsc.parallel_loop(0, CH // 16, unroll=UNROLL)(scan_body)
                if t + 2 < nchunks:
                    copies[b] = pltpu.async_copy(
                        src.at[pl.ds((t + 2) * CH, CH)], buf, sems[b])
            pltpu.sync_copy(acc, dst.at[pl.ds(r0, TILE_ROWS), pl.ds(c0, HALF)])


def _adj_sc(ei0, ei1):
    mesh = plsc.VectorSubcoreMesh(core_axis_name="c", subcore_axis_name="s")
    f = functools.partial(
        pl.kernel,
        out_type=(jax.ShapeDtypeStruct((N, N), jnp.float32),
                  jax.ShapeDtypeStruct((N, N), jnp.float32)),
        mesh=mesh,
        scratch_types=[
            pltpu.VMEM((TILE_ROWS, HALF), jnp.float32),
            pltpu.VMEM((CH,), jnp.int32),
            pltpu.VMEM((CH,), jnp.int32),
            pltpu.SemaphoreType.DMA,
            pltpu.SemaphoreType.DMA,
        ],
        compiler_params=pltpu.CompilerParams(use_tc_tiling_on_sc=False,
                                             needs_layout_passes=False),
    )(_adj_sc_body)
    f0, f1 = _pack_edges(ei0, ei1)
    return f(f0.reshape(E), f1.reshape(E))


def _stack_body(prec, a_ref, x_ref, dos_ref, dis_ref, w0_ref, b0_ref, w1_ref,
                b1_ref, o_ref, ot_ref, h1_scr):
    p = pl.program_id(0)
    j = pl.program_id(1)

    @pl.when(p == 0)
    def _():
        hs = x_ref[...] * dos_ref[...]
        agg = lax.dot_general(a_ref[...], hs, (((0,), (0,)), ((), ())),
                              preferred_element_type=jnp.float32, precision=prec)
        agg = agg * dis_ref[pl.ds(j * RB, RB), :]
        o = jnp.dot(agg, w0_ref[...], preferred_element_type=jnp.float32,
                    precision=prec) + b0_ref[...]
        h1_scr[pl.ds(j * RB, RB), :] = jnp.maximum(o, 0.0)

    @pl.when(p == 1)
    def _():
        hs = h1_scr[...] * dos_ref[...]
        agg = lax.dot_general(a_ref[...], hs, (((0,), (0,)), ((), ())),
                              preferred_element_type=jnp.float32, precision=prec)
        agg = agg * dis_ref[pl.ds(j * RB, RB), :]
        o = jnp.dot(agg, w1_ref[...], preferred_element_type=jnp.float32,
                    precision=prec) + b1_ref[...]
        o_ref[...] = o
        ot_ref[...] = o.T


def _gcn_stack(a, x, dos, dis, w0, b0, w1, b1, prec=_HI):
    hin = x.shape[1]
    hmid = w0.shape[1]
    hout = w1.shape[1]
    return pl.pallas_call(
        functools.partial(_stack_body, prec),
        grid=(2, G),
        in_specs=[
            pl.BlockSpec((N, RB), lambda p, j: (0, j)),
            pl.BlockSpec((N, hin), lambda p, j: (0, 0)),
            pl.BlockSpec((N, 1), lambda p, j: (0, 0)),
            pl.BlockSpec((N, 1), lambda p, j: (0, 0)),
            pl.BlockSpec((hin, hmid), lambda p, j: (0, 0)),
            pl.BlockSpec((1, hmid), lambda p, j: (0, 0)),
            pl.BlockSpec((hmid, hout), lambda p, j: (0, 0)),
            pl.BlockSpec((1, hout), lambda p, j: (0, 0)),
        ],
        out_specs=(pl.BlockSpec((RB, hout), lambda p, j: (j, 0)),
                   pl.BlockSpec((hout, RB), lambda p, j: (0, j))),
        out_shape=(jax.ShapeDtypeStruct((N, hout), jnp.float32),
                   jax.ShapeDtypeStruct((hout, N), jnp.float32)),
        scratch_shapes=[pltpu.VMEM((N, hmid), jnp.float32)],
    )(a, x, dos, dis, w0, b0.reshape(1, hmid), w1, b1.reshape(1, hout))


def _fuse_body(f0_ref, f1_ref, c0_ref, c1_ref, z_ref):
    zz = (jnp.dot(f0_ref[...], c0_ref[...], preferred_element_type=jnp.float32)
          + jnp.dot(f1_ref[...], c1_ref[...], preferred_element_type=jnp.float32))
    zz = zz - jnp.max(zz, axis=1, keepdims=True)
    e = jnp.exp(zz)
    z_ref[...] = e / jnp.sum(e, axis=1, keepdims=True)


def _fuse(f0, f1, c0, c1):
    return pl.pallas_call(
        _fuse_body,
        out_shape=jax.ShapeDtypeStruct(f0.shape, jnp.float32),
    )(f0, f1, c0, c1)


def _gfn1_body(a0_ref, a1_ref, w_ref, b_ref, t_ref, do0_ref, di0_ref,
               do1_ref, di1_ref):
    j = pl.program_id(0)
    a0 = a0_ref[...]
    a1 = a1_ref[...]
    do0_ref[...] = lax.rsqrt(jnp.clip(jnp.sum(a0, axis=1, keepdims=True), 1.0, None))
    do1_ref[...] = lax.rsqrt(jnp.clip(jnp.sum(a1, axis=1, keepdims=True), 1.0, None))

    @pl.when(j == 0)
    def _():
        di0_ref[...] = jnp.zeros_like(di0_ref)
        di1_ref[...] = jnp.zeros_like(di1_ref)

    di0_ref[...] += jnp.sum(a0, axis=0, keepdims=True)
    di1_ref[...] += jnp.sum(a1, axis=0, keepdims=True)

    @pl.when(j == G - 1)
    def _():
        di0_ref[...] = lax.rsqrt(jnp.clip(di0_ref[...], 1.0, None))
        di1_ref[...] = lax.rsqrt(jnp.clip(di1_ref[...], 1.0, None))

    adj = jnp.minimum(a0, 1.0) + jnp.minimum(a1, 1.0)
    t = jnp.dot(adj, w_ref[...], preferred_element_type=jnp.float32) + b_ref[...]
    t_ref[...] = jnp.maximum(t, 0.0)


def _gfn1(a0, a1, w, b):
    return pl.pallas_call(
        _gfn1_body,
        grid=(G,),
        in_specs=[
            pl.BlockSpec((RB, N), lambda j: (j, 0)),
            pl.BlockSpec((RB, N), lambda j: (j, 0)),
            pl.BlockSpec((N, N // 2), lambda j: (0, 0)),
            pl.BlockSpec((1, N // 2), lambda j: (0, 0)),
        ],
        out_specs=(
            pl.BlockSpec((RB, N // 2), lambda j: (j, 0)),
            pl.BlockSpec((RB, 1), lambda j: (j, 0)),
            pl.BlockSpec((1, N), lambda j: (0, 0)),
            pl.BlockSpec((RB, 1), lambda j: (j, 0)),
            pl.BlockSpec((1, N), lambda j: (0, 0)),
        ),
        out_shape=(jax.ShapeDtypeStruct((N, N // 2), jnp.float32),
                   jax.ShapeDtypeStruct((N, 1), jnp.float32),
                   jax.ShapeDtypeStruct((1, N), jnp.float32),
                   jax.ShapeDtypeStruct((N, 1), jnp.float32),
                   jax.ShapeDtypeStruct((1, N), jnp.float32)),
    )(a0, a1, w, b.reshape(1, N // 2))


def _gfn2_body(t_ref, w_ref, b_ref, o_ref):
    v = jnp.dot(t_ref[...], w_ref[...], preferred_element_type=jnp.float32) + b_ref[...]
    o_ref[...] = jnp.where(v > jnp.float32(0.4), 1.0, 0.0)


def _gfn2(t, w, b):
    return pl.pallas_call(
        _gfn2_body,
        grid=(G,),
        in_specs=[
            pl.BlockSpec((RB, N // 2), lambda j: (j, 0)),
            pl.BlockSpec((N // 2, N), lambda j: (0, 0)),
            pl.BlockSpec((1, N), lambda j: (0, 0)),
        ],
        out_specs=pl.BlockSpec((RB, N), lambda j: (j, 0)),
        out_shape=jax.ShapeDtypeStruct((N, N), jnp.float32),
    )(t, w, b.reshape(1, N))


def _aloop_body(fr_ref, fc_ref, al_ref, d_ref):
    i = pl.program_id(0)
    j = pl.program_id(1)
    sym = fr_ref[...] + fc_ref[...].T
    a = jnp.where(sym != 0.0, 1.0, 0.0)
    row_ids = lax.broadcasted_iota(jnp.int32, (RB, RB), 0) + i * RB
    col_ids = lax.broadcasted_iota(jnp.int32, (RB, RB), 1) + j * RB
    a = a + jnp.where(row_ids == col_ids, 1.0, 0.0)
    al_ref[...] = a

    @pl.when(j == 0)
    def _():
        d_ref[...] = jnp.zeros_like(d_ref)

    d_ref[...] += jnp.sum(a, axis=1, keepdims=True)

    @pl.when(j == G - 1)
    def _():
        d_ref[...] = lax.rsqrt(jnp.clip(d_ref[...], 1.0, None))


def _aloop(fused):
    return pl.pallas_call(
        _aloop_body,
        grid=(G, G),
        in_specs=[
            pl.BlockSpec((RB, RB), lambda i, j: (i, j)),
            pl.BlockSpec((RB, RB), lambda i, j: (j, i)),
        ],
        out_specs=(
            pl.BlockSpec((RB, RB), lambda i, j: (i, j)),
            pl.BlockSpec((RB, 1), lambda i, j: (i, 0)),
        ),
        out_shape=(jax.ShapeDtypeStruct((N, N), jnp.float32),
                   jax.ShapeDtypeStruct((N, 1), jnp.float32)),
    )(fused, fused)


def _dec_body(hb_ref, ht_ref, o_ref):
    o_ref[...] = jnp.dot(hb_ref[...], ht_ref[...],
                         preferred_element_type=jnp.float32)


def _decode(h, ht):
    hd = h.shape[1]
    return pl.pallas_call(
        _dec_body,
        grid=(G,),
        in_specs=[
            pl.BlockSpec((RB, hd), lambda j: (j, 0)),
            pl.BlockSpec((hd, N), lambda j: (0, 0)),
        ],
        out_specs=pl.BlockSpec((RB, N), lambda j: (j, 0)),
        out_shape=jax.ShapeDtypeStruct((N, N), jnp.float32),
    )(h, ht)


def kernel(x0, x1, edge_index0, edge_index1, Wv00, bv00, Wv01, bv01,
           Wv10, bv10, Wv11, bv11, F0, F1, GW1, Gb1, GW2, Gb2,
           Wm0, bm0, Wm1, bm1):
    a0, a1 = _adj_sc(edge_index0, edge_index1)
    t, do0, di0, do1, di1 = _gfn1(a0, a1, GW1, Gb1)
    di0 = di0.reshape(N, 1)
    di1 = di1.reshape(N, 1)
    f0, _ = _gcn_stack(a0, x0, do0, di0, Wv00, bv00, Wv01, bv01,
                       prec=lax.Precision.DEFAULT)
    f1, _ = _gcn_stack(a1, x1, do1, di1, Wv10, bv10, Wv11, bv11,
                       prec=lax.Precision.DEFAULT)
    z = _fuse(f0, f1, F0, F1)
    fused = _gfn2(t, GW2, Gb2)
    al, scons = _aloop(fused)
    h, ht = _gcn_stack(al, z, scons, scons, Wm0, bm0, Wm1, bm1,
                       prec=lax.Precision.DEFAULT)
    adj_rec = _decode(h, ht)
    return (fused, adj_rec, h)
